# baseline (speedup 1.0000x reference)
import functools

import jax
import jax.numpy as jnp
from jax import lax
from jax.experimental import pallas as pl
from jax.experimental.pallas import tpu as pltpu
from jax.experimental.pallas import tpu_sc as plsc

_NUM_CORES = 2
_NUM_SUBCORES = 16
_LANES = 16
_CHUNK = 4096


@functools.partial(jax.jit, static_argnums=(4,))
def _run(phase, tsw_pad, packed_flat, p2pack, hop):
    batch, seq_len = phase.shape
    tsw_w = tsw_pad.shape[1]
    flat_len = packed_flat.shape[0]

    frames_per_chunk = _CHUNK // hop
    assert frames_per_chunk % 2 == 0
    n_pairs = seq_len // (2 * _CHUNK)
    vecs_per_frame = hop // _LANES

    mesh = plsc.VectorSubcoreMesh(
        core_axis_name="c", subcore_axis_name="s",
        num_cores=_NUM_CORES, num_subcores=_NUM_SUBCORES)

    @functools.partial(
        pl.kernel,
        out_type=jax.ShapeDtypeStruct((batch, seq_len), jnp.float32),
        mesh=mesh,
        scratch_types=[
            pltpu.VMEM((flat_len,), jnp.int32),
            pltpu.VMEM_SHARED((flat_len,), jnp.int32),
            pltpu.VMEM((tsw_w,), jnp.float32),
            pltpu.VMEM((tsw_w,), jnp.int32),
            pltpu.VMEM((tsw_w,), jnp.float32),
            pltpu.VMEM((hop,), jnp.int32),
            pltpu.VMEM((_CHUNK,), jnp.float32),
            pltpu.VMEM((_CHUNK,), jnp.float32),
            pltpu.VMEM((_CHUNK,), jnp.float32),
            pltpu.VMEM((_CHUNK,), jnp.float32),
            pltpu.SemaphoreType.DMA,
            pltpu.SemaphoreType.DMA,
            pltpu.SemaphoreType.DMA,
            pltpu.SemaphoreType.DMA,
            pltpu.SemaphoreType.DMA,
        ],
        compiler_params=pltpu.CompilerParams(needs_layout_passes=False),
    )
    def run(phase_hbm, tsw_hbm, table_hbm, p2p_hbm, out_hbm,
            tab_v, shr_v, tsw_v, row_v, frac_v, p2p_v,
            ph0_v, ph1_v, ou0_v, ou1_v, si0, si1, so0, so1, st):
        sid = lax.axis_index("s")
        wid = sid * _NUM_CORES + lax.axis_index("c")

        stripe = flat_len // _NUM_SUBCORES
        ssl = pl.ds(sid * stripe, stripe)
        table_dma = pltpu.async_copy(table_hbm.at[ssl], shr_v.at[ssl], st)
        pltpu.sync_copy(p2p_hbm, p2p_v)
        pltpu.sync_copy(tsw_hbm.at[wid], tsw_v)

        t_minus_1 = float(_NUM_TABLES - 1)
        for j in range(tsw_w // _LANES):
            sl = pl.ds(j * _LANES, _LANES)
            w = tsw_v[sl] * t_minus_1
            r = jnp.clip(w.astype(jnp.int32), 0, _NUM_TABLES - 2)
            row_v[sl] = r * _TABLE_LEN
            frac_v[sl] = w - r.astype(jnp.float32)

        table_dma.wait()
        plsc.subcore_barrier()
        pltpu.sync_copy(shr_v, tab_v)

        def compute(ph_v, out_v, c):
            @plsc.parallel_loop(0, frames_per_chunk, 1, unroll=2)
            def frame_body(fl):
                f = c * frames_per_chunk + fl
                fvec = jnp.full((_LANES,), f, dtype=jnp.int32)
                rfb = plsc.load_gather(row_v, [fvec])
                qf = plsc.load_gather(frac_v, [fvec])
                rcb = plsc.load_gather(row_v, [fvec + 1])
                qc = plsc.load_gather(frac_v, [fvec + 1])
                ifmt = plsc.PackFormat.INTERLEAVED
                wf1 = plsc.pack(qf, qf, format=ifmt)
                wc1 = plsc.pack(qc, qc, format=ifmt)
                rf1b = rfb + _TABLE_LEN
                rc1b = rcb + _TABLE_LEN
                base = fl * hop

                @plsc.parallel_loop(0, hop, _LANES, unroll=2)
                def _(k):
                    sl = pl.ds(base + k, _LANES)
                    ksl = pl.ds(k, _LANES)
                    x = ph_v[sl] * float(_TABLE_LEN)
                    i0 = x.astype(jnp.int32)
                    px = x - i0.astype(jnp.float32)
                    ompx = 1.0 - px
                    g0 = plsc.load_gather(tab_v, [rfb + i0])
                    g1 = plsc.load_gather(tab_v, [rf1b + i0])
                    g2 = plsc.load_gather(tab_v, [rcb + i0])
                    g3 = plsc.load_gather(tab_v, [rc1b + i0])
                    p0 = plsc.bitcast(g0, jnp.bfloat16)
                    p1 = plsc.bitcast(g1, jnp.bfloat16)
                    p2_ = plsc.bitcast(g2, jnp.bfloat16)
                    p3 = plsc.bitcast(g3, jnp.bfloat16)
                    sfp = p0 + (p1 - p0) * wf1
                    scp = p2_ + (p3 - p2_) * wc1
                    p2k = plsc.bitcast(p2p_v[ksl], jnp.bfloat16)
                    acc = sfp + (scp - sfp) * p2k
                    u, v = plsc.unpack(acc, format=ifmt)
                    out_v[sl] = u * ompx + v * px

        def start_in(buf, sem, c):
            pltpu.async_copy(phase_hbm.at[wid, pl.ds(c * _CHUNK, _CHUNK)],
                             buf, sem)

        def start_out(buf, sem, c):
            pltpu.async_copy(buf, out_hbm.at[wid, pl.ds(c * _CHUNK, _CHUNK)],
                             sem)

        def wait_in(buf, sem):
            pltpu.make_async_copy(phase_hbm.at[wid, pl.ds(0, _CHUNK)],
                                  buf, sem).wait()

        def wait_out(buf, sem):
            pltpu.make_async_copy(buf, out_hbm.at[wid, pl.ds(0, _CHUNK)],
                                  sem).wait()

        start_in(ph0_v, si0, 0)
        start_in(ph1_v, si1, 1)

        def pair_body(c2, carry):
            c0 = 2 * c2

            wait_in(ph0_v, si0)

            @pl.when(c2 > 0)
            def _():
                wait_out(ou0_v, so0)

            compute(ph0_v, ou0_v, c0)
            start_out(ou0_v, so0, c0)

            @pl.when(c2 < n_pairs - 1)
            def _():
                start_in(ph0_v, si0, c0 + 2)

            wait_in(ph1_v, si1)

            @pl.when(c2 > 0)
            def _():
                wait_out(ou1_v, so1)

            compute(ph1_v, ou1_v, c0 + 1)
            start_out(ou1_v, so1, c0 + 1)

            @pl.when(c2 < n_pairs - 1)
            def _():
                start_in(ph1_v, si1, c0 + 3)

            return carry

        lax.fori_loop(0, n_pairs, pair_body, 0)
        wait_out(ou0_v, so0)
        wait_out(ou1_v, so1)

    return run(phase, tsw_pad, packed_flat, p2pack)


_NUM_TABLES = 100
_TABLE_LEN = 1024


def _pack_pair_bits(a_bf16, b_bf16):
    lo = jax.lax.bitcast_convert_type(a_bf16, jnp.uint16).astype(jnp.uint32)
    hi = jax.lax.bitcast_convert_type(b_bf16, jnp.uint16).astype(jnp.uint32)
    return jax.lax.bitcast_convert_type(lo | (hi << 16), jnp.int32)


def kernel(wrapped_phase, table_select_weight, table, hop_size):
    batch, seq_len = wrapped_phase.shape
    n_frames_p1 = table_select_weight.shape[1]
    hop = seq_len // (n_frames_p1 - 1)

    assert batch == _NUM_CORES * _NUM_SUBCORES
    assert table.shape == (_NUM_TABLES, _TABLE_LEN)
    assert hop % _LANES == 0 and _CHUNK % hop == 0
    assert seq_len % (2 * _CHUNK) == 0

    tb = table.astype(jnp.bfloat16)
    packed_flat = _pack_pair_bits(tb, jnp.roll(tb, -1, axis=1)).reshape(-1)

    tsw_w = -(-n_frames_p1 // _LANES) * _LANES
    tsw_pad = jnp.pad(table_select_weight, ((0, 0), (0, tsw_w - n_frames_p1)))

    p2 = (jnp.arange(hop, dtype=wrapped_phase.dtype) / hop_size)
    p2pack = _pack_pair_bits(p2.astype(jnp.bfloat16), p2.astype(jnp.bfloat16))
    return _run(wrapped_phase, tsw_pad, packed_flat, p2pack, hop)


# speedup vs baseline: 1.0665x; 1.0665x over previous
import functools

import jax
import jax.numpy as jnp
from jax import lax
from jax.experimental import pallas as pl
from jax.experimental.pallas import tpu as pltpu
from jax.experimental.pallas import tpu_sc as plsc

_NUM_CORES = 2
_NUM_SUBCORES = 16
_LANES = 16
_CHUNK = 4096


@functools.partial(jax.jit, static_argnums=(5,))
def _run(phase, tsw_pad, packed_flat, p2pack, omp2pack, hop):
    batch, seq_len = phase.shape
    tsw_w = tsw_pad.shape[1]
    flat_len = packed_flat.shape[0]

    frames_per_chunk = _CHUNK // hop
    assert frames_per_chunk % 2 == 0
    n_pairs = seq_len // (2 * _CHUNK)
    vecs_per_frame = hop // _LANES

    mesh = plsc.VectorSubcoreMesh(
        core_axis_name="c", subcore_axis_name="s",
        num_cores=_NUM_CORES, num_subcores=_NUM_SUBCORES)

    @functools.partial(
        pl.kernel,
        out_type=jax.ShapeDtypeStruct((batch, seq_len), jnp.float32),
        mesh=mesh,
        scratch_types=[
            pltpu.VMEM((flat_len,), jnp.int32),
            pltpu.VMEM_SHARED((flat_len,), jnp.int32),
            pltpu.VMEM((tsw_w,), jnp.float32),
            pltpu.VMEM((tsw_w,), jnp.int32),
            pltpu.VMEM((tsw_w,), jnp.float32),
            pltpu.VMEM((hop,), jnp.int32),
            pltpu.VMEM((hop,), jnp.int32),
            pltpu.VMEM((_CHUNK,), jnp.float32),
            pltpu.VMEM((_CHUNK,), jnp.float32),
            pltpu.VMEM((_CHUNK,), jnp.float32),
            pltpu.VMEM((_CHUNK,), jnp.float32),
            pltpu.SemaphoreType.DMA,
            pltpu.SemaphoreType.DMA,
            pltpu.SemaphoreType.DMA,
            pltpu.SemaphoreType.DMA,
            pltpu.SemaphoreType.DMA,
        ],
        compiler_params=pltpu.CompilerParams(needs_layout_passes=False),
    )
    def run(phase_hbm, tsw_hbm, table_hbm, p2p_hbm, omp2p_hbm, out_hbm,
            tab_v, shr_v, tsw_v, row_v, frac_v, p2p_v, omp2p_v,
            ph0_v, ph1_v, ou0_v, ou1_v, si0, si1, so0, so1, st):
        sid = lax.axis_index("s")
        wid = sid * _NUM_CORES + lax.axis_index("c")

        stripe = flat_len // _NUM_SUBCORES
        ssl = pl.ds(sid * stripe, stripe)
        pltpu.async_copy(phase_hbm.at[wid, pl.ds(0, _CHUNK)], ph0_v, si0)
        pltpu.async_copy(phase_hbm.at[wid, pl.ds(_CHUNK, _CHUNK)], ph1_v, si1)
        table_dma = pltpu.async_copy(table_hbm.at[ssl], shr_v.at[ssl], st)
        pltpu.sync_copy(p2p_hbm, p2p_v)
        pltpu.sync_copy(omp2p_hbm, omp2p_v)
        pltpu.sync_copy(tsw_hbm.at[wid], tsw_v)

        t_minus_1 = float(_NUM_TABLES - 1)
        for j in range(tsw_w // _LANES):
            sl = pl.ds(j * _LANES, _LANES)
            w = tsw_v[sl] * t_minus_1
            r = jnp.clip(w.astype(jnp.int32), 0, _NUM_TABLES - 2)
            row_v[sl] = r * _TABLE_LEN
            frac_v[sl] = w - r.astype(jnp.float32)

        table_dma.wait()
        plsc.subcore_barrier()
        pltpu.sync_copy(shr_v, tab_v)

        def compute(ph_v, out_v, c):
            @plsc.parallel_loop(0, frames_per_chunk, 1, unroll=2)
            def frame_body(fl):
                f = c * frames_per_chunk + fl
                fvec = jnp.full((_LANES,), f, dtype=jnp.int32)
                rfb = plsc.load_gather(row_v, [fvec])
                qf = plsc.load_gather(frac_v, [fvec])
                rcb = plsc.load_gather(row_v, [fvec + 1])
                qc = plsc.load_gather(frac_v, [fvec + 1])
                omqf = 1.0 - qf
                omqc = 1.0 - qc
                ifmt = plsc.PackFormat.INTERLEAVED
                wf0 = plsc.pack(omqf, omqf, format=ifmt)
                wf1 = plsc.pack(qf, qf, format=ifmt)
                wc0 = plsc.pack(omqc, omqc, format=ifmt)
                wc1 = plsc.pack(qc, qc, format=ifmt)
                rf1b = rfb + _TABLE_LEN
                rc1b = rcb + _TABLE_LEN
                base = fl * hop

                @plsc.parallel_loop(0, hop, _LANES, unroll=2)
                def _(k):
                    sl = pl.ds(base + k, _LANES)
                    ksl = pl.ds(k, _LANES)
                    x = ph_v[sl] * float(_TABLE_LEN)
                    i0 = x.astype(jnp.int32)
                    px = x - i0.astype(jnp.float32)
                    ompx = 1.0 - px
                    g0 = plsc.load_gather(tab_v, [rfb + i0])
                    g1 = plsc.load_gather(tab_v, [rf1b + i0])
                    g2 = plsc.load_gather(tab_v, [rcb + i0])
                    g3 = plsc.load_gather(tab_v, [rc1b + i0])
                    p0 = plsc.bitcast(g0, jnp.bfloat16)
                    p1 = plsc.bitcast(g1, jnp.bfloat16)
                    p2_ = plsc.bitcast(g2, jnp.bfloat16)
                    p3 = plsc.bitcast(g3, jnp.bfloat16)
                    sfp = p0 * wf0 + p1 * wf1
                    scp = p2_ * wc0 + p3 * wc1
                    p2k = plsc.bitcast(p2p_v[ksl], jnp.bfloat16)
                    omp2k = plsc.bitcast(omp2p_v[ksl], jnp.bfloat16)
                    acc = sfp * omp2k + scp * p2k
                    u, v = plsc.unpack(acc, format=ifmt)
                    out_v[sl] = u * ompx + v * px

        def start_in(buf, sem, c):
            pltpu.async_copy(phase_hbm.at[wid, pl.ds(c * _CHUNK, _CHUNK)],
                             buf, sem)

        def start_out(buf, sem, c):
            pltpu.async_copy(buf, out_hbm.at[wid, pl.ds(c * _CHUNK, _CHUNK)],
                             sem)

        def wait_in(buf, sem):
            pltpu.make_async_copy(phase_hbm.at[wid, pl.ds(0, _CHUNK)],
                                  buf, sem).wait()

        def wait_out(buf, sem):
            pltpu.make_async_copy(buf, out_hbm.at[wid, pl.ds(0, _CHUNK)],
                                  sem).wait()

        def pair_body(c2, carry):
            c0 = 2 * c2

            wait_in(ph0_v, si0)

            @pl.when(c2 > 0)
            def _():
                wait_out(ou0_v, so0)

            compute(ph0_v, ou0_v, c0)
            start_out(ou0_v, so0, c0)

            @pl.when(c2 < n_pairs - 1)
            def _():
                start_in(ph0_v, si0, c0 + 2)

            wait_in(ph1_v, si1)

            @pl.when(c2 > 0)
            def _():
                wait_out(ou1_v, so1)

            compute(ph1_v, ou1_v, c0 + 1)
            start_out(ou1_v, so1, c0 + 1)

            @pl.when(c2 < n_pairs - 1)
            def _():
                start_in(ph1_v, si1, c0 + 3)

            return carry

        lax.fori_loop(0, n_pairs, pair_body, 0)
        wait_out(ou0_v, so0)
        wait_out(ou1_v, so1)

    return run(phase, tsw_pad, packed_flat, p2pack, omp2pack)


_NUM_TABLES = 100
_TABLE_LEN = 1024


def _pack_pair_bits(a_bf16, b_bf16):
    lo = jax.lax.bitcast_convert_type(a_bf16, jnp.uint16).astype(jnp.uint32)
    hi = jax.lax.bitcast_convert_type(b_bf16, jnp.uint16).astype(jnp.uint32)
    return jax.lax.bitcast_convert_type(lo | (hi << 16), jnp.int32)


def kernel(wrapped_phase, table_select_weight, table, hop_size):
    batch, seq_len = wrapped_phase.shape
    n_frames_p1 = table_select_weight.shape[1]
    hop = seq_len // (n_frames_p1 - 1)

    assert batch == _NUM_CORES * _NUM_SUBCORES
    assert table.shape == (_NUM_TABLES, _TABLE_LEN)
    assert hop % _LANES == 0 and _CHUNK % hop == 0
    assert seq_len % (2 * _CHUNK) == 0

    tb = table.astype(jnp.bfloat16)
    packed_flat = _pack_pair_bits(tb, jnp.roll(tb, -1, axis=1)).reshape(-1)

    tsw_w = -(-n_frames_p1 // _LANES) * _LANES
    tsw_pad = jnp.pad(table_select_weight, ((0, 0), (0, tsw_w - n_frames_p1)))

    p2 = (jnp.arange(hop, dtype=wrapped_phase.dtype) / hop_size)
    p2pack = _pack_pair_bits(p2.astype(jnp.bfloat16), p2.astype(jnp.bfloat16))
    omp2 = (1.0 - p2).astype(jnp.bfloat16)
    omp2pack = _pack_pair_bits(omp2, omp2)

    return _run(wrapped_phase, tsw_pad, packed_flat, p2pack, omp2pack, hop)
